# baseline (speedup 1.0000x reference)
import functools

import jax
import jax.numpy as jnp
from jax import lax
from jax.experimental import pallas as pl
from jax.experimental.pallas import tpu as pltpu
from jax.experimental.pallas import tpu_sc as plsc

N = 10000
E = 160000
D = 256
NPAD = 10240
NC, NS = 2, 16
NT = NC * NS
EPT = E // NT
FG = 4
NGRP = D // (NT * FG)
CE = 16000
NCH = E // CE

_mesh = plsc.VectorSubcoreMesh(
    core_axis_name="c", subcore_axis_name="s", num_cores=NC, num_subcores=NS)
_CP = pltpu.CompilerParams(needs_layout_passes=False)


@functools.partial(
    pl.kernel,
    out_type=jax.ShapeDtypeStruct((NT, NPAD), jnp.float32),
    mesh=_mesh,
    compiler_params=_CP,
    scratch_types=[
        pltpu.VMEM((NPAD + 16,), jnp.float32),
        pltpu.VMEM((EPT + 8,), jnp.int32),
    ],
)
def _deg_kernel(dst_hbm, degp_hbm, counts, dbuf):
    w = lax.axis_index("c") * NS + lax.axis_index("s")

    def _zero(i, _):
        counts[pl.ds(i * 16, 16)] = jnp.zeros((16,), jnp.float32)
        return 0
    lax.fori_loop(0, (NPAD + 16) // 16, _zero, 0)

    dbuf[pl.ds(EPT - 8, 16)] = jnp.full((16,), NPAD, jnp.int32)
    pltpu.sync_copy(dst_hbm.at[pl.ds(w * EPT, EPT)], dbuf.at[pl.ds(0, EPT)])

    ones = jnp.ones((16,), jnp.float32)

    def _count(i, _):
        dv = dbuf[pl.ds(i * 16, 16)]
        plsc.addupdate_scatter(counts, [dv], ones)
        return 0
    lax.fori_loop(0, (EPT + 8) // 16, _count, 0)

    pltpu.sync_copy(counts.at[pl.ds(0, NPAD)], degp_hbm.at[w])


@functools.partial(
    pl.kernel,
    out_type=jax.ShapeDtypeStruct((D, NPAD), jnp.float32),
    mesh=_mesh,
    compiler_params=_CP,
    scratch_types=[
        pltpu.VMEM((FG, NPAD), jnp.float32),
        pltpu.VMEM((FG, NPAD), jnp.float32),
        pltpu.VMEM((CE,), jnp.int32),
        pltpu.VMEM((CE,), jnp.int32),
    ],
)
def _agg_kernel(ht_hbm, src_hbm, dst_hbm, aggt_hbm, hbuf, acc, sbuf, dbuf):
    w = lax.axis_index("c") * NS + lax.axis_index("s")

    for g in range(NGRP):
        fbase = w * (FG * NGRP) + g * FG
        pltpu.sync_copy(ht_hbm.at[pl.ds(fbase, FG)], hbuf)
        pltpu.sync_copy(ht_hbm.at[pl.ds(fbase, FG)], acc)

        def _chunk(ci, _):
            pltpu.sync_copy(src_hbm.at[pl.ds(ci * CE, CE)], sbuf)
            pltpu.sync_copy(dst_hbm.at[pl.ds(ci * CE, CE)], dbuf)

            @plsc.parallel_loop(0, CE // 16, unroll=16)
            def _vec(i):
                sv = sbuf[pl.ds(i * 16, 16)]
                dv = dbuf[pl.ds(i * 16, 16)]
                for fl in range(FG):
                    fidx = jnp.full((16,), fl, jnp.int32)
                    vals = plsc.load_gather(hbuf, [fidx, sv])
                    plsc.addupdate_scatter(acc, [fidx, dv], vals)
            return 0
        lax.fori_loop(0, NCH, _chunk, 0)

        pltpu.sync_copy(acc, aggt_hbm.at[pl.ds(fbase, FG)])


BLK = NPAD // 4


def _dis_row(degp_blk):
    deg = jnp.sum(degp_blk, axis=0) + 1.0
    return (1.0 / jnp.sqrt(deg))[None, :]


def _mm1_body(x_ref, w_ref, degp_ref, o_ref):
    dis = _dis_row(degp_ref[...])
    o_ref[...] = lax.dot_general(
        w_ref[...], x_ref[...], (((0,), (1,)), ((), ())),
        preferred_element_type=jnp.float32) * dis


def _mid_body(a_ref, degp_ref, b_ref, w_ref, o_ref):
    dis = _dis_row(degp_ref[...])
    y = jnp.maximum(a_ref[...] * dis + b_ref[...], 0.0)
    o_ref[...] = lax.dot_general(
        w_ref[...], y, (((0,), (0,)), ((), ())),
        preferred_element_type=jnp.float32) * dis


def _final_body(a_ref, degp_ref, b_ref, o_ref):
    dis = _dis_row(degp_ref[...])
    o_ref[...] = jnp.maximum(a_ref[...] * dis + b_ref[...], 0.0).T


_rows_spec = pl.BlockSpec((BLK, D), lambda i: (i, 0))
_t_spec = pl.BlockSpec((D, BLK), lambda i: (0, i))
_w_spec = pl.BlockSpec((D, D), lambda i: (0, 0))
_degp_spec = pl.BlockSpec((NT, BLK), lambda i: (0, i))
_bcol_spec = pl.BlockSpec((D, 1), lambda i: (0, 0))
_t_sds = jax.ShapeDtypeStruct((D, NPAD), jnp.float32)

_mm1 = pl.pallas_call(
    _mm1_body, grid=(NPAD // BLK,),
    in_specs=[_rows_spec, _w_spec, _degp_spec],
    out_specs=_t_spec, out_shape=_t_sds)

_mid = pl.pallas_call(
    _mid_body, grid=(NPAD // BLK,),
    in_specs=[_t_spec, _degp_spec, _bcol_spec, _w_spec],
    out_specs=_t_spec, out_shape=_t_sds)

_final = pl.pallas_call(
    _final_body, grid=(NPAD // BLK,),
    in_specs=[_t_spec, _degp_spec, _bcol_spec],
    out_specs=_rows_spec,
    out_shape=jax.ShapeDtypeStruct((NPAD, D), jnp.float32))


def kernel(x, A, W1, b1, W2, b2):
    src = A[0]
    dst = A[1]
    x_pad = jnp.pad(x, ((0, NPAD - N), (0, 0)))
    b1c = b1.reshape(D, 1)
    b2c = b2.reshape(D, 1)

    degp = _deg_kernel(dst)
    h1t = _mm1(x_pad, W1, degp)
    agg1t = _agg_kernel(h1t, src, dst)
    h2t = _mid(agg1t, degp, b1c, W2)
    agg2t = _agg_kernel(h2t, src, dst)
    out = _final(agg2t, degp, b2c)
    return out[:N]


# speedup vs baseline: 1.0550x; 1.0550x over previous
import functools

import jax
import jax.numpy as jnp
from jax import lax
from jax.experimental import pallas as pl
from jax.experimental.pallas import tpu as pltpu
from jax.experimental.pallas import tpu_sc as plsc

N = 10000
E = 160000
D = 256
NPAD = 10240
NC, NS = 2, 16
NT = NC * NS
EPT = E // NT
FG = 4
NGRP = D // (NT * FG)
CE = 16000
NCH = E // CE

_mesh = plsc.VectorSubcoreMesh(
    core_axis_name="c", subcore_axis_name="s", num_cores=NC, num_subcores=NS)
_CP = pltpu.CompilerParams(needs_layout_passes=False)


@functools.partial(
    pl.kernel,
    out_type=jax.ShapeDtypeStruct((NT, NPAD), jnp.float32),
    mesh=_mesh,
    compiler_params=_CP,
    scratch_types=[
        pltpu.VMEM((NPAD + 16,), jnp.float32),
        pltpu.VMEM((EPT + 8,), jnp.int32),
    ],
)
def _deg_kernel(dst_hbm, degp_hbm, counts, dbuf):
    w = lax.axis_index("c") * NS + lax.axis_index("s")

    def _zero(i, _):
        counts[pl.ds(i * 16, 16)] = jnp.zeros((16,), jnp.float32)
        return 0
    lax.fori_loop(0, (NPAD + 16) // 16, _zero, 0)

    dbuf[pl.ds(EPT - 8, 16)] = jnp.full((16,), NPAD, jnp.int32)
    pltpu.sync_copy(dst_hbm.at[pl.ds(w * EPT, EPT)], dbuf.at[pl.ds(0, EPT)])

    ones = jnp.ones((16,), jnp.float32)

    def _count(i, _):
        dv = dbuf[pl.ds(i * 16, 16)]
        plsc.addupdate_scatter(counts, [dv], ones)
        return 0
    lax.fori_loop(0, (EPT + 8) // 16, _count, 0)

    pltpu.sync_copy(counts.at[pl.ds(0, NPAD)], degp_hbm.at[w])


@functools.partial(
    pl.kernel,
    out_type=jax.ShapeDtypeStruct((D, NPAD), jnp.float32),
    mesh=_mesh,
    compiler_params=_CP,
    scratch_types=[
        pltpu.VMEM((FG * NPAD,), jnp.float32),
        pltpu.VMEM((FG * NPAD,), jnp.float32),
        pltpu.VMEM((CE,), jnp.int32),
        pltpu.VMEM((CE,), jnp.int32),
    ],
)
def _agg_kernel(ht_hbm, src_hbm, dst_hbm, aggt_hbm, hbuf, acc, sbuf, dbuf):
    w = lax.axis_index("c") * NS + lax.axis_index("s")

    for g in range(NGRP):
        fbase = w * (FG * NGRP) + g * FG
        for fl in range(FG):
            pltpu.sync_copy(ht_hbm.at[fbase + fl], hbuf.at[pl.ds(fl * NPAD, NPAD)])
            pltpu.sync_copy(ht_hbm.at[fbase + fl], acc.at[pl.ds(fl * NPAD, NPAD)])

        def _chunk(ci, _):
            pltpu.sync_copy(src_hbm.at[pl.ds(ci * CE, CE)], sbuf)
            pltpu.sync_copy(dst_hbm.at[pl.ds(ci * CE, CE)], dbuf)

            @plsc.parallel_loop(0, CE // 16, unroll=8)
            def _vec(i):
                sv = sbuf[pl.ds(i * 16, 16)]
                dv = dbuf[pl.ds(i * 16, 16)]
                for fl in range(FG):
                    gi = sv + (fl * NPAD) if fl else sv
                    di = dv + (fl * NPAD) if fl else dv
                    vals = plsc.load_gather(hbuf, [gi])
                    plsc.addupdate_scatter(acc, [di], vals)
            return 0
        lax.fori_loop(0, NCH, _chunk, 0)

        for fl in range(FG):
            pltpu.sync_copy(acc.at[pl.ds(fl * NPAD, NPAD)],
                            aggt_hbm.at[fbase + fl])


BLK = NPAD // 4


def _dis_row(degp_blk):
    deg = jnp.sum(degp_blk, axis=0) + 1.0
    return (1.0 / jnp.sqrt(deg))[None, :]


def _mm1_body(x_ref, w_ref, degp_ref, o_ref):
    dis = _dis_row(degp_ref[...])
    o_ref[...] = lax.dot_general(
        w_ref[...], x_ref[...], (((0,), (1,)), ((), ())),
        preferred_element_type=jnp.float32) * dis


def _mid_body(a_ref, degp_ref, b_ref, w_ref, o_ref):
    dis = _dis_row(degp_ref[...])
    y = jnp.maximum(a_ref[...] * dis + b_ref[...], 0.0)
    o_ref[...] = lax.dot_general(
        w_ref[...], y, (((0,), (0,)), ((), ())),
        preferred_element_type=jnp.float32) * dis


def _final_body(a_ref, degp_ref, b_ref, o_ref):
    dis = _dis_row(degp_ref[...])
    o_ref[...] = jnp.maximum(a_ref[...] * dis + b_ref[...], 0.0).T


_rows_spec = pl.BlockSpec((BLK, D), lambda i: (i, 0))
_t_spec = pl.BlockSpec((D, BLK), lambda i: (0, i))
_w_spec = pl.BlockSpec((D, D), lambda i: (0, 0))
_degp_spec = pl.BlockSpec((NT, BLK), lambda i: (0, i))
_bcol_spec = pl.BlockSpec((D, 1), lambda i: (0, 0))
_t_sds = jax.ShapeDtypeStruct((D, NPAD), jnp.float32)

_mm1 = pl.pallas_call(
    _mm1_body, grid=(NPAD // BLK,),
    in_specs=[_rows_spec, _w_spec, _degp_spec],
    out_specs=_t_spec, out_shape=_t_sds)

_mid = pl.pallas_call(
    _mid_body, grid=(NPAD // BLK,),
    in_specs=[_t_spec, _degp_spec, _bcol_spec, _w_spec],
    out_specs=_t_spec, out_shape=_t_sds)

_final = pl.pallas_call(
    _final_body, grid=(NPAD // BLK,),
    in_specs=[_t_spec, _degp_spec, _bcol_spec],
    out_specs=_rows_spec,
    out_shape=jax.ShapeDtypeStruct((NPAD, D), jnp.float32))


def kernel(x, A, W1, b1, W2, b2):
    src = A[0]
    dst = A[1]
    x_pad = jnp.pad(x, ((0, NPAD - N), (0, 0)))
    b1c = b1.reshape(D, 1)
    b2c = b2.reshape(D, 1)

    degp = _deg_kernel(dst)
    h1t = _mm1(x_pad, W1, degp)
    agg1t = _agg_kernel(h1t, src, dst)
    h2t = _mid(agg1t, degp, b1c, W2)
    agg2t = _agg_kernel(h2t, src, dst)
    out = _final(agg2t, degp, b2c)
    return out[:N]


# speedup vs baseline: 1.1118x; 1.0538x over previous
import functools

import jax
import jax.numpy as jnp
from jax import lax
from jax.experimental import pallas as pl
from jax.experimental.pallas import tpu as pltpu
from jax.experimental.pallas import tpu_sc as plsc

N = 10000
E = 160000
D = 256
NPAD = 10240
NC, NS = 2, 16
NT = NC * NS
EPT = E // NT
FG = 4
NGRP = D // (NT * FG)
CE = 16000
NCH = E // CE

_mesh = plsc.VectorSubcoreMesh(
    core_axis_name="c", subcore_axis_name="s", num_cores=NC, num_subcores=NS)
_CP = pltpu.CompilerParams(needs_layout_passes=False)


@functools.partial(
    pl.kernel,
    out_type=jax.ShapeDtypeStruct((NT, NPAD), jnp.float32),
    mesh=_mesh,
    compiler_params=_CP,
    scratch_types=[
        pltpu.VMEM((NPAD + 16,), jnp.float32),
        pltpu.VMEM((EPT + 8,), jnp.int32),
    ],
)
def _deg_kernel(dst_hbm, degp_hbm, counts, dbuf):
    w = lax.axis_index("c") * NS + lax.axis_index("s")

    def _zero(i, _):
        counts[pl.ds(i * 16, 16)] = jnp.zeros((16,), jnp.float32)
        return 0
    lax.fori_loop(0, (NPAD + 16) // 16, _zero, 0)

    dbuf[pl.ds(EPT - 8, 16)] = jnp.full((16,), NPAD, jnp.int32)
    pltpu.sync_copy(dst_hbm.at[pl.ds(w * EPT, EPT)], dbuf.at[pl.ds(0, EPT)])

    ones = jnp.ones((16,), jnp.float32)

    def _count(i, _):
        dv = dbuf[pl.ds(i * 16, 16)]
        plsc.addupdate_scatter(counts, [dv], ones)
        return 0
    lax.fori_loop(0, (EPT + 8) // 16, _count, 0)

    pltpu.sync_copy(counts.at[pl.ds(0, NPAD)], degp_hbm.at[w])


@functools.partial(
    pl.kernel,
    out_type=jax.ShapeDtypeStruct((D, NPAD), jnp.float32),
    mesh=_mesh,
    compiler_params=_CP,
    scratch_types=[
        pltpu.VMEM((FG, NPAD), jnp.float32),
        pltpu.VMEM((FG, NPAD), jnp.float32),
        pltpu.VMEM((CE,), jnp.int32),
        pltpu.VMEM((CE,), jnp.int32),
    ],
)
def _agg_kernel(ht_hbm, src_hbm, dst_hbm, aggt_hbm, hbuf, acc, sbuf, dbuf):
    w = lax.axis_index("c") * NS + lax.axis_index("s")

    for g in range(NGRP):
        fbase = w * (FG * NGRP) + g * FG
        pltpu.sync_copy(ht_hbm.at[pl.ds(fbase, FG)], hbuf)
        pltpu.sync_copy(ht_hbm.at[pl.ds(fbase, FG)], acc)

        def _chunk(ci, _):
            pltpu.sync_copy(src_hbm.at[pl.ds(ci * CE, CE)], sbuf)
            pltpu.sync_copy(dst_hbm.at[pl.ds(ci * CE, CE)], dbuf)

            @plsc.parallel_loop(0, CE // 16, unroll=4)
            def _vec(i):
                sv = sbuf[pl.ds(i * 16, 16)]
                dv = dbuf[pl.ds(i * 16, 16)]
                for fl in range(FG):
                    fidx = jnp.full((16,), fl, jnp.int32)
                    vals = plsc.load_gather(hbuf, [fidx, sv])
                    plsc.addupdate_scatter(acc, [fidx, dv], vals)
            return 0
        lax.fori_loop(0, NCH, _chunk, 0)

        pltpu.sync_copy(acc, aggt_hbm.at[pl.ds(fbase, FG)])


BLK = NPAD // 4


def _dis_row(degp_blk):
    deg = jnp.sum(degp_blk, axis=0) + 1.0
    return (1.0 / jnp.sqrt(deg))[None, :]


def _mm1_body(x_ref, w_ref, degp_ref, o_ref):
    dis = _dis_row(degp_ref[...])
    o_ref[...] = lax.dot_general(
        w_ref[...], x_ref[...], (((0,), (1,)), ((), ())),
        preferred_element_type=jnp.float32) * dis


def _mid_body(a_ref, degp_ref, b_ref, w_ref, o_ref):
    dis = _dis_row(degp_ref[...])
    y = jnp.maximum(a_ref[...] * dis + b_ref[...], 0.0)
    o_ref[...] = lax.dot_general(
        w_ref[...], y, (((0,), (0,)), ((), ())),
        preferred_element_type=jnp.float32) * dis


def _final_body(a_ref, degp_ref, b_ref, o_ref):
    dis = _dis_row(degp_ref[...])
    o_ref[...] = jnp.maximum(a_ref[...] * dis + b_ref[...], 0.0).T


_rows_spec = pl.BlockSpec((BLK, D), lambda i: (i, 0))
_t_spec = pl.BlockSpec((D, BLK), lambda i: (0, i))
_w_spec = pl.BlockSpec((D, D), lambda i: (0, 0))
_degp_spec = pl.BlockSpec((NT, BLK), lambda i: (0, i))
_bcol_spec = pl.BlockSpec((D, 1), lambda i: (0, 0))
_t_sds = jax.ShapeDtypeStruct((D, NPAD), jnp.float32)

_mm1 = pl.pallas_call(
    _mm1_body, grid=(NPAD // BLK,),
    in_specs=[_rows_spec, _w_spec, _degp_spec],
    out_specs=_t_spec, out_shape=_t_sds)

_mid = pl.pallas_call(
    _mid_body, grid=(NPAD // BLK,),
    in_specs=[_t_spec, _degp_spec, _bcol_spec, _w_spec],
    out_specs=_t_spec, out_shape=_t_sds)

_final = pl.pallas_call(
    _final_body, grid=(NPAD // BLK,),
    in_specs=[_t_spec, _degp_spec, _bcol_spec],
    out_specs=_rows_spec,
    out_shape=jax.ShapeDtypeStruct((NPAD, D), jnp.float32))


def kernel(x, A, W1, b1, W2, b2):
    src = A[0]
    dst = A[1]
    x_pad = jnp.pad(x, ((0, NPAD - N), (0, 0)))
    b1c = b1.reshape(D, 1)
    b2c = b2.reshape(D, 1)

    degp = _deg_kernel(dst)
    h1t = _mm1(x_pad, W1, degp)
    agg1t = _agg_kernel(h1t, src, dst)
    h2t = _mid(agg1t, degp, b1c, W2)
    agg2t = _agg_kernel(h2t, src, dst)
    out = _final(agg2t, degp, b2c)
    return out[:N]


# speedup vs baseline: 1.3428x; 1.2078x over previous
import functools

import jax
import jax.numpy as jnp
from jax import lax
from jax.experimental import pallas as pl
from jax.experimental.pallas import tpu as pltpu
from jax.experimental.pallas import tpu_sc as plsc

N = 10000
E = 160000
D = 256
NPAD = 10240
NC, NS = 2, 16
NT = NC * NS
EPT = E // NT
FG = 4
NGRP = D // (NT * FG)
CE = 8000
NPAIR = E // (2 * CE)

_mesh = plsc.VectorSubcoreMesh(
    core_axis_name="c", subcore_axis_name="s", num_cores=NC, num_subcores=NS)
_CP = pltpu.CompilerParams(needs_layout_passes=False)


@functools.partial(
    pl.kernel,
    out_type=jax.ShapeDtypeStruct((NT, NPAD), jnp.float32),
    mesh=_mesh,
    compiler_params=_CP,
    scratch_types=[
        pltpu.VMEM((NPAD + 16,), jnp.float32),
        pltpu.VMEM((EPT + 8,), jnp.int32),
    ],
)
def _deg_kernel(dst_hbm, degp_hbm, counts, dbuf):
    w = lax.axis_index("c") * NS + lax.axis_index("s")

    def _zero(i, _):
        counts[pl.ds(i * 16, 16)] = jnp.zeros((16,), jnp.float32)
        return 0
    lax.fori_loop(0, (NPAD + 16) // 16, _zero, 0)

    dbuf[pl.ds(EPT - 8, 16)] = jnp.full((16,), NPAD, jnp.int32)
    pltpu.sync_copy(dst_hbm.at[pl.ds(w * EPT, EPT)], dbuf.at[pl.ds(0, EPT)])

    ones = jnp.ones((16,), jnp.float32)

    def _count(i, _):
        dv = dbuf[pl.ds(i * 16, 16)]
        plsc.addupdate_scatter(counts, [dv], ones)
        return 0
    lax.fori_loop(0, (EPT + 8) // 16, _count, 0)

    pltpu.sync_copy(counts.at[pl.ds(0, NPAD)], degp_hbm.at[w])


@functools.partial(
    pl.kernel,
    out_type=jax.ShapeDtypeStruct((D, NPAD), jnp.float32),
    mesh=_mesh,
    compiler_params=_CP,
    scratch_types=[
        pltpu.VMEM((FG, NPAD), jnp.float32),
        pltpu.VMEM((FG, NPAD), jnp.float32),
        pltpu.VMEM((CE,), jnp.int32),
        pltpu.VMEM((CE,), jnp.int32),
        pltpu.VMEM((CE,), jnp.int32),
        pltpu.VMEM((CE,), jnp.int32),
        pltpu.SemaphoreType.DMA,
        pltpu.SemaphoreType.DMA,
    ],
)
def _agg_kernel(ht_hbm, src_hbm, dst_hbm, aggt_hbm, hbuf, acc, sbuf0, sbuf1,
                dbuf0, dbuf1, sem0, sem1):
    w = lax.axis_index("c") * NS + lax.axis_index("s")
    slots = ((sbuf0, dbuf0, sem0), (sbuf1, dbuf1, sem1))

    def _start(ci, slot):
        sb, db, sem = slots[slot]
        pltpu.async_copy(src_hbm.at[pl.ds(ci * CE, CE)], sb, sem)
        pltpu.async_copy(dst_hbm.at[pl.ds(ci * CE, CE)], db, sem)

    def _wait(slot):
        sb, db, sem = slots[slot]
        pltpu.make_async_copy(src_hbm.at[pl.ds(0, CE)], sb, sem).wait()
        pltpu.make_async_copy(dst_hbm.at[pl.ds(0, CE)], db, sem).wait()

    def _process(slot):
        sb, db, _ = slots[slot]

        @plsc.parallel_loop(0, CE // 16, unroll=4)
        def _vec(i):
            sv = sb[pl.ds(i * 16, 16)]
            dv = db[pl.ds(i * 16, 16)]
            for fl in range(FG):
                fidx = jnp.full((16,), fl, jnp.int32)
                vals = plsc.load_gather(hbuf, [fidx, sv])
                plsc.addupdate_scatter(acc, [fidx, dv], vals)

    for g in range(NGRP):
        fbase = w * (FG * NGRP) + g * FG
        _start(0, 0)
        pltpu.sync_copy(ht_hbm.at[pl.ds(fbase, FG)], hbuf)
        pltpu.sync_copy(ht_hbm.at[pl.ds(fbase, FG)], acc)

        def _pair(p, _):
            _start(2 * p + 1, 1)
            _wait(0)
            _process(0)

            @pl.when(p < NPAIR - 1)
            def _():
                _start(2 * p + 2, 0)
            _wait(1)
            _process(1)
            return 0
        lax.fori_loop(0, NPAIR, _pair, 0)

        pltpu.sync_copy(acc, aggt_hbm.at[pl.ds(fbase, FG)])


BLK = NPAD // 4


def _dis_row(degp_blk):
    deg = jnp.sum(degp_blk, axis=0) + 1.0
    return (1.0 / jnp.sqrt(deg))[None, :]


def _mm1_body(x_ref, w_ref, degp_ref, o_ref):
    dis = _dis_row(degp_ref[...])
    o_ref[...] = lax.dot_general(
        w_ref[...], x_ref[...], (((0,), (1,)), ((), ())),
        preferred_element_type=jnp.float32) * dis


def _mid_body(a_ref, degp_ref, b_ref, w_ref, o_ref):
    dis = _dis_row(degp_ref[...])
    y = jnp.maximum(a_ref[...] * dis + b_ref[...], 0.0)
    o_ref[...] = lax.dot_general(
        w_ref[...], y, (((0,), (0,)), ((), ())),
        preferred_element_type=jnp.float32) * dis


def _final_body(a_ref, degp_ref, b_ref, o_ref):
    dis = _dis_row(degp_ref[...])
    o_ref[...] = jnp.maximum(a_ref[...] * dis + b_ref[...], 0.0).T


_rows_spec = pl.BlockSpec((BLK, D), lambda i: (i, 0))
_t_spec = pl.BlockSpec((D, BLK), lambda i: (0, i))
_w_spec = pl.BlockSpec((D, D), lambda i: (0, 0))
_degp_spec = pl.BlockSpec((NT, BLK), lambda i: (0, i))
_bcol_spec = pl.BlockSpec((D, 1), lambda i: (0, 0))
_t_sds = jax.ShapeDtypeStruct((D, NPAD), jnp.float32)

_mm1 = pl.pallas_call(
    _mm1_body, grid=(NPAD // BLK,),
    in_specs=[_rows_spec, _w_spec, _degp_spec],
    out_specs=_t_spec, out_shape=_t_sds)

_mid = pl.pallas_call(
    _mid_body, grid=(NPAD // BLK,),
    in_specs=[_t_spec, _degp_spec, _bcol_spec, _w_spec],
    out_specs=_t_spec, out_shape=_t_sds)

_final = pl.pallas_call(
    _final_body, grid=(NPAD // BLK,),
    in_specs=[_t_spec, _degp_spec, _bcol_spec],
    out_specs=_rows_spec,
    out_shape=jax.ShapeDtypeStruct((NPAD, D), jnp.float32))


def kernel(x, A, W1, b1, W2, b2):
    src = A[0]
    dst = A[1]
    x_pad = jnp.pad(x, ((0, NPAD - N), (0, 0)))
    b1c = b1.reshape(D, 1)
    b2c = b2.reshape(D, 1)

    degp = _deg_kernel(dst)
    h1t = _mm1(x_pad, W1, degp)
    agg1t = _agg_kernel(h1t, src, dst)
    h2t = _mid(agg1t, degp, b1c, W2)
    agg2t = _agg_kernel(h2t, src, dst)
    out = _final(agg2t, degp, b2c)
    return out[:N]
